# baseline (speedup 1.0000x reference)
import jax
import jax.numpy as jnp
from jax.experimental import pallas as pl

B, N, IN_DIM = 8, 512, 64
HEADS, HEAD_DIM = 4, 16


def _gat_kernel(x_ref, adj_ref, w_ref, asrc_ref, adst_ref, out_ref):
    xb = x_ref[0]
    adjb = adj_ref[0]

    h = jnp.dot(xb, w_ref[...], preferred_element_type=jnp.float32)

    mask = adjb > 0.5
    has_edge = jnp.any(mask)
    row = jax.lax.broadcasted_iota(jnp.int32, (N, N), 0)
    col = jax.lax.broadcasted_iota(jnp.int32, (N, N), 1)
    mask = mask | ((row == col) & jnp.logical_not(has_edge))
    b01 = jnp.where(mask, 1.0, 0.0)

    ones = jnp.full((N, 1), 1.0, dtype=jnp.float32)

    for hh in range(HEADS):
        h_head = h[:, hh * HEAD_DIM:(hh + 1) * HEAD_DIM]
        a_s = asrc_ref[hh:hh + 1, :]
        a_d = adst_ref[hh:hh + 1, :]
        s = jax.lax.dot_general(h_head, a_s, (((1,), (1,)), ((), ())),
                                preferred_element_type=jnp.float32)
        d = jax.lax.dot_general(a_d, h_head, (((1,), (1,)), ((), ())),
                                preferred_element_type=jnp.float32)
        mm = jnp.max(s) + d
        mrow = jnp.maximum(mm, 0.2 * mm)
        e = s + d
        l = jnp.maximum(e, 0.2 * e)
        ex = jnp.exp(l - mrow) * b01
        hcat = jnp.concatenate([h_head, ones], axis=1)
        agg = jax.lax.dot_general(ex, hcat, (((0,), (0,)), ((), ())),
                                  preferred_element_type=jnp.float32)
        out = agg[:, :HEAD_DIM] / (agg[:, HEAD_DIM:HEAD_DIM + 1] + 1e-16)
        out_ref[0, :, hh * HEAD_DIM:(hh + 1) * HEAD_DIM] = out


@jax.jit
def kernel(x, adj, W, a_src, a_dst):
    w_flat = W.reshape(IN_DIM, HEADS * HEAD_DIM)
    return pl.pallas_call(
        _gat_kernel,
        grid=(B,),
        in_specs=[
            pl.BlockSpec((1, N, IN_DIM), lambda b: (b, 0, 0)),
            pl.BlockSpec((1, N, N), lambda b: (b, 0, 0)),
            pl.BlockSpec((IN_DIM, HEADS * HEAD_DIM), lambda b: (0, 0)),
            pl.BlockSpec((HEADS, HEAD_DIM), lambda b: (0, 0)),
            pl.BlockSpec((HEADS, HEAD_DIM), lambda b: (0, 0)),
        ],
        out_specs=pl.BlockSpec((1, N, HEADS * HEAD_DIM), lambda b: (b, 0, 0)),
        out_shape=jax.ShapeDtypeStruct((B, N, HEADS * HEAD_DIM), jnp.float32),
    )(x, adj, w_flat, a_src, a_dst)


# speedup vs baseline: 1.5381x; 1.5381x over previous
import jax
import jax.numpy as jnp
from jax.experimental import pallas as pl

B, N, IN_DIM = 8, 512, 64
HEADS, HEAD_DIM = 4, 16
OUT_DIM = HEADS * HEAD_DIM
AGG_W = OUT_DIM + HEADS


def _gat_kernel(x_ref, adj_ref, w_ref, asrc_ref, adst_ref, s68_ref, out_ref):
    xb = x_ref[0]
    adjb = adj_ref[0]

    h = jnp.dot(xb, w_ref[...], preferred_element_type=jnp.float32)

    mask = adjb > 0.5
    has_edge = jnp.any(mask)
    row = jax.lax.broadcasted_iota(jnp.int32, (N, N), 0)
    col = jax.lax.broadcasted_iota(jnp.int32, (N, N), 1)
    mask = mask | ((row == col) & jnp.logical_not(has_edge))
    b01 = jnp.where(mask, 1.0, 0.0).astype(jnp.bfloat16)

    es = jnp.dot(h, asrc_ref[...], preferred_element_type=jnp.float32)
    edT = jax.lax.dot_general(adst_ref[...], h, (((0,), (1,)), ((), ())),
                              preferred_element_type=jnp.float32)
    Ms = jnp.max(es, axis=0, keepdims=True)
    es2 = 0.2 * es

    ones4 = jnp.full((N, HEADS), 1.0, dtype=jnp.float32)
    rhs = jnp.concatenate([h, ones4], axis=1).astype(jnp.bfloat16)

    lane = jax.lax.broadcasted_iota(jnp.int32, (N, AGG_W), 1)
    total = jnp.zeros((N, AGG_W), dtype=jnp.float32)
    for hh in range(HEADS):
        s = es[:, hh:hh + 1]
        s2 = es2[:, hh:hh + 1]
        dr = edT[hh:hh + 1, :]
        mm = Ms[:, hh:hh + 1] + dr
        mrow = jnp.maximum(mm, 0.2 * mm)
        d1 = dr - mrow
        d2 = 0.2 * dr - mrow
        t = jnp.maximum(s + d1, s2 + d2)
        exb = jnp.exp(t).astype(jnp.bfloat16) * b01
        agg = jax.lax.dot_general(exb, rhs, (((0,), (0,)), ((), ())),
                                  preferred_element_type=jnp.float32)
        head_lanes = ((lane >= hh * HEAD_DIM) & (lane < (hh + 1) * HEAD_DIM)
                      ) | (lane == OUT_DIM + hh)
        total = total + jnp.where(head_lanes, agg, 0.0)

    den_guarded = jnp.where(lane >= OUT_DIM, total, 1.0)
    recip = 1.0 / (den_guarded + 1e-16)
    scale = jnp.dot(recip, s68_ref[...], preferred_element_type=jnp.float32)
    out_ref[0] = total[:, :OUT_DIM] * scale


@jax.jit
def kernel(x, adj, W, a_src, a_dst):
    w_flat = W.reshape(IN_DIM, OUT_DIM)
    eyeh = jnp.eye(HEADS, dtype=jnp.float32)
    asrc_bd = (a_src[:, :, None] * eyeh[:, None, :]).reshape(IN_DIM, HEADS)
    adst_bd = (a_dst[:, :, None] * eyeh[:, None, :]).reshape(IN_DIM, HEADS)
    kidx = jnp.arange(AGG_W)[:, None]
    jidx = jnp.arange(OUT_DIM)[None, :]
    s68 = (kidx == OUT_DIM + jidx // HEAD_DIM).astype(jnp.float32)
    return pl.pallas_call(
        _gat_kernel,
        grid=(B,),
        in_specs=[
            pl.BlockSpec((1, N, IN_DIM), lambda b: (b, 0, 0)),
            pl.BlockSpec((1, N, N), lambda b: (b, 0, 0)),
            pl.BlockSpec((IN_DIM, OUT_DIM), lambda b: (0, 0)),
            pl.BlockSpec((IN_DIM, HEADS), lambda b: (0, 0)),
            pl.BlockSpec((IN_DIM, HEADS), lambda b: (0, 0)),
            pl.BlockSpec((AGG_W, OUT_DIM), lambda b: (0, 0)),
        ],
        out_specs=pl.BlockSpec((1, N, OUT_DIM), lambda b: (b, 0, 0)),
        out_shape=jax.ShapeDtypeStruct((B, N, OUT_DIM), jnp.float32),
    )(x, adj, w_flat, asrc_bd, adst_bd, s68)


# speedup vs baseline: 1.6146x; 1.0498x over previous
import numpy as np

import jax
import jax.numpy as jnp
from jax.experimental import pallas as pl

B, N, IN_DIM = 8, 512, 64
HEADS, HEAD_DIM = 4, 16
OUT_DIM = HEADS * HEAD_DIM
AGG_W = OUT_DIM + HEADS
BPS = 2

_S68 = np.zeros((AGG_W, OUT_DIM), dtype=np.float32)
for _h in range(HEADS):
    _S68[OUT_DIM + _h, _h * HEAD_DIM:(_h + 1) * HEAD_DIM] = 1.0


def _gat_one(xb, adjb, w, wsrc, wdst, s68):
    h = jnp.dot(xb, w, preferred_element_type=jnp.float32)

    mask = adjb > 0.5
    has_edge = jnp.any(mask)
    row = jax.lax.broadcasted_iota(jnp.int32, (N, N), 0)
    col = jax.lax.broadcasted_iota(jnp.int32, (N, N), 1)
    mask = mask | ((row == col) & jnp.logical_not(has_edge))
    b01 = jnp.where(mask, 1.0, 0.0).astype(jnp.bfloat16)

    es = jnp.dot(xb, wsrc, preferred_element_type=jnp.float32)
    edT = jax.lax.dot_general(wdst, xb, (((0,), (1,)), ((), ())),
                              preferred_element_type=jnp.float32)
    Ms = jnp.max(es, axis=0, keepdims=True)
    es2 = 0.2 * es

    ones4 = jnp.full((N, HEADS), 1.0, dtype=jnp.float32)
    rhs = jnp.concatenate([h, ones4], axis=1).astype(jnp.bfloat16)

    lane = jax.lax.broadcasted_iota(jnp.int32, (N, AGG_W), 1)
    total = jnp.zeros((N, AGG_W), dtype=jnp.float32)
    for hh in range(HEADS):
        s = es[:, hh:hh + 1]
        s2 = es2[:, hh:hh + 1]
        dr = edT[hh:hh + 1, :]
        mm = Ms[:, hh:hh + 1] + dr
        mrow = jnp.maximum(mm, 0.2 * mm)
        d1 = dr - mrow
        d2 = 0.2 * dr - mrow
        t = jnp.maximum(s + d1, s2 + d2)
        exb = jnp.exp(t).astype(jnp.bfloat16) * b01
        agg = jax.lax.dot_general(exb, rhs, (((0,), (0,)), ((), ())),
                                  preferred_element_type=jnp.float32)
        head_lanes = ((lane >= hh * HEAD_DIM) & (lane < (hh + 1) * HEAD_DIM)
                      ) | (lane == OUT_DIM + hh)
        total = total + jnp.where(head_lanes, agg, 0.0)

    den_guarded = jnp.where(lane >= OUT_DIM, total, 1.0)
    recip = 1.0 / (den_guarded + 1e-16)
    scale = jnp.dot(recip, s68, preferred_element_type=jnp.float32)
    return total[:, :OUT_DIM] * scale


def _gat_kernel(x_ref, adj_ref, w_ref, wsrc_ref, wdst_ref, s68_ref, out_ref):
    for bb in range(BPS):
        out_ref[bb] = _gat_one(x_ref[bb], adj_ref[bb], w_ref[...],
                               wsrc_ref[...], wdst_ref[...], s68_ref[...])


@jax.jit
def kernel(x, adj, W, a_src, a_dst):
    w_flat = W.reshape(IN_DIM, OUT_DIM)
    wsrc = jnp.einsum('dhf,hf->dh', W, a_src)
    wdst = jnp.einsum('dhf,hf->dh', W, a_dst)
    s68 = jnp.asarray(_S68)
    return pl.pallas_call(
        _gat_kernel,
        grid=(B // BPS,),
        in_specs=[
            pl.BlockSpec((BPS, N, IN_DIM), lambda b: (b, 0, 0)),
            pl.BlockSpec((BPS, N, N), lambda b: (b, 0, 0)),
            pl.BlockSpec((IN_DIM, OUT_DIM), lambda b: (0, 0)),
            pl.BlockSpec((IN_DIM, HEADS), lambda b: (0, 0)),
            pl.BlockSpec((IN_DIM, HEADS), lambda b: (0, 0)),
            pl.BlockSpec((AGG_W, OUT_DIM), lambda b: (0, 0)),
        ],
        out_specs=pl.BlockSpec((BPS, N, OUT_DIM), lambda b: (b, 0, 0)),
        out_shape=jax.ShapeDtypeStruct((B, N, OUT_DIM), jnp.float32),
    )(x, adj, w_flat, wsrc, wdst, s68)
